# baseline (speedup 1.0000x reference)
import functools

import jax
import jax.numpy as jnp
from jax import lax
from jax.experimental import pallas as pl
from jax.experimental.pallas import tpu as pltpu
from jax.experimental.pallas import tpu_sc as plsc

BATCH = 1024
SEQ = 50
EMB = 16
NL = 16
TOT = BATCH * SEQ


def _crf_body(g_ref, tags_ref, w_ref, b_ref, trans_ref, transT_ref, out_ref):
    Wm = w_ref[...]
    bias = b_ref[...]
    trans = trans_ref[...]
    transT = transT_ref[...]
    Et = jnp.exp(transT)
    lab_iota = lax.broadcasted_iota(jnp.int32, (NL, BATCH), 0)
    tr_start = transT[:, 0:1]
    tr_end = trans[:, 1:2]

    def emit(t):
        e = g_ref[pl.ds(t * BATCH, BATCH), :]
        em = lax.dot_general(Wm, e, (((1,), (1,)), ((), ())),
                             preferred_element_type=jnp.float32,
                             precision=lax.Precision.HIGHEST)
        return em + bias

    def selmask(t):
        tg = tags_ref[pl.ds(t, 1), :]
        return (lab_iota == tg).astype(jnp.float32)

    em0 = emit(0)
    sel0 = selmask(0)
    alphas0 = tr_start + em0
    acc0 = sel0 * (em0 + tr_start)

    def step(t, carry):
        alphas, acc, selp = carry
        em = emit(t)
        sel = selmask(t)
        m = jnp.max(alphas, axis=0, keepdims=True)
        p = jnp.exp(alphas - m)
        s = lax.dot_general(Et, p, (((1,), (0,)), ((), ())),
                            preferred_element_type=jnp.float32,
                            precision=lax.Precision.HIGHEST)
        alphas = em + m + jnp.log(s)
        tsel = lax.dot_general(transT, selp, (((1,), (0,)), ((), ())),
                               preferred_element_type=jnp.float32,
                               precision=lax.Precision.HIGHEST)
        acc = acc + sel * (em + tsel)
        return alphas, acc, sel

    alphas, acc, sel_last = lax.fori_loop(1, SEQ, step, (alphas0, acc0, sel0))
    acc = acc + sel_last * tr_end
    end = alphas + tr_end
    m = jnp.max(end, axis=0, keepdims=True)
    part = m + jnp.log(jnp.sum(jnp.exp(end - m), axis=0, keepdims=True))
    out_ref[...] = (jnp.sum(part) - jnp.sum(acc)).reshape(1, 1)


def _sc_gather(table, idx):
    info = plsc.get_sparse_core_info()
    nc, ns = info.num_cores, info.num_subcores
    nw = nc * ns
    bpw = TOT // nw
    csz = 400
    nch = bpw // csz

    mesh = plsc.VectorSubcoreMesh(core_axis_name="c", subcore_axis_name="s")

    @functools.partial(
        pl.kernel,
        mesh=mesh,
        out_type=jax.ShapeDtypeStruct((TOT, EMB), jnp.float32),
        scratch_types=[
            pltpu.VMEM((bpw,), jnp.int32),
            pltpu.SemaphoreType.DMA,
            pltpu.SemaphoreType.DMA,
        ],
    )
    def gk(table_hbm, idx_hbm, out_hbm, idx_v, sem_i, sem_g):
        wid = lax.axis_index("s") * nc + lax.axis_index("c")
        base = wid * bpw
        pltpu.async_copy(idx_hbm.at[pl.ds(base, bpw)], idx_v, sem_i).wait()

        def issue(v, _):
            vec = idx_v[pl.ds(v * 16, 16)]
            for j in range(16):
                pltpu.make_async_copy(
                    table_hbm.at[pl.ds(vec[j], 1), :],
                    out_hbm.at[pl.ds(base + v * 16 + j, 1), :],
                    sem_g,
                ).start()
            return 0

        lax.fori_loop(0, bpw // 16, issue, 0)

        def drain(i, _):
            pltpu.make_async_copy(
                table_hbm.at[pl.ds(0, 1), :],
                out_hbm.at[pl.ds(base + i, 1), :],
                sem_g,
            ).wait()
            return 0

        lax.fori_loop(0, bpw, drain, 0, unroll=8)

    return gk(table, idx)


def kernel(x, tags, mask, embed_table, W, b, transitions):
    idx = jnp.transpose(x).reshape(-1)
    g = _sc_gather(embed_table, idx)
    out = pl.pallas_call(
        _crf_body,
        out_shape=jax.ShapeDtypeStruct((1, 1), jnp.float32),
    )(g, jnp.transpose(tags), W, b.reshape(NL, 1), transitions,
      jnp.transpose(transitions))
    return out[0, 0]


# speedup vs baseline: 3.0407x; 3.0407x over previous
import functools

import jax
import jax.numpy as jnp
from jax import lax
from jax.experimental import pallas as pl
from jax.experimental.pallas import tpu as pltpu
from jax.experimental.pallas import tpu_sc as plsc

BATCH = 1024
SEQ = 50
EMB = 16
NL = 16
TOT = BATCH * SEQ


def _crf_body(g_ref, tags_ref, w_ref, b_ref, trans_ref, transT_ref, out_ref):
    Wm = w_ref[...]
    bias = b_ref[...]
    trans = trans_ref[...]
    transT = transT_ref[...]
    Et = jnp.exp(transT)
    lab_iota = lax.broadcasted_iota(jnp.int32, (NL, BATCH), 0)
    tr_start = transT[:, 0:1]
    tr_end = trans[:, 1:2]

    def emit(t):
        e = g_ref[pl.ds(t * BATCH, BATCH), :]
        em = lax.dot_general(Wm, e, (((1,), (1,)), ((), ())),
                             preferred_element_type=jnp.float32,
                             precision=lax.Precision.HIGHEST)
        return em + bias

    def selmask(t):
        tg = tags_ref[pl.ds(t, 1), :]
        return (lab_iota == tg).astype(jnp.float32)

    em0 = emit(0)
    sel0 = selmask(0)
    alphas0 = tr_start + em0
    acc0 = sel0 * (em0 + tr_start)

    def step(t, carry):
        alphas, acc, selp = carry
        em = emit(t)
        sel = selmask(t)
        m = jnp.max(alphas, axis=0, keepdims=True)
        p = jnp.exp(alphas - m)
        s = lax.dot_general(Et, p, (((1,), (0,)), ((), ())),
                            preferred_element_type=jnp.float32,
                            precision=lax.Precision.HIGHEST)
        alphas = em + m + jnp.log(s)
        tsel = lax.dot_general(transT, selp, (((1,), (0,)), ((), ())),
                               preferred_element_type=jnp.float32,
                               precision=lax.Precision.HIGHEST)
        acc = acc + sel * (em + tsel)
        return alphas, acc, sel

    alphas, acc, sel_last = lax.fori_loop(1, SEQ, step, (alphas0, acc0, sel0))
    acc = acc + sel_last * tr_end
    end = alphas + tr_end
    m = jnp.max(end, axis=0, keepdims=True)
    part = m + jnp.log(jnp.sum(jnp.exp(end - m), axis=0, keepdims=True))
    out_ref[...] = (jnp.sum(part) - jnp.sum(acc)).reshape(1, 1)


def _sc_gather(table, idx):
    info = plsc.get_sparse_core_info()
    nc, ns = info.num_cores, info.num_subcores
    nw = nc * ns
    bpw = TOT // nw
    csz = 400
    nch = bpw // csz
    ngrp = csz // 16

    mesh = plsc.VectorSubcoreMesh(core_axis_name="c", subcore_axis_name="s")

    @functools.partial(
        pl.kernel,
        mesh=mesh,
        out_type=jax.ShapeDtypeStruct((TOT, EMB), jnp.float32),
        scratch_types=[
            pltpu.VMEM((bpw,), jnp.int32),
            pltpu.VMEM((csz, EMB), jnp.float32),
            pltpu.SemaphoreType.DMA,
            pltpu.SemaphoreType.DMA,
        ],
    )
    def gk(table_hbm, idx_hbm, out_hbm, idx_v, rows_v, sem_i, sem_g):
        wid = lax.axis_index("s") * nc + lax.axis_index("c")
        base = wid * bpw
        pltpu.async_copy(idx_hbm.at[pl.ds(base, bpw)], idx_v, sem_i).wait()

        def issue16(ch, v):
            vec = idx_v[pl.ds(ch * csz + v * 16, 16)]
            for j in range(16):
                pltpu.make_async_copy(
                    table_hbm.at[pl.ds(vec[j], 1), :],
                    rows_v.at[pl.ds(v * 16 + j, 1), :],
                    sem_g,
                ).start()

        def drain16(v):
            for j in range(16):
                pltpu.make_async_copy(
                    table_hbm.at[pl.ds(0, 1), :],
                    rows_v.at[pl.ds(v * 16 + j, 1), :],
                    sem_g,
                ).wait()

        def chunk(ch, _):
            issue16(ch, 0)

            def grp(v, _):
                issue16(ch, v)
                drain16(v - 1)
                return 0

            lax.fori_loop(1, ngrp, grp, 0)
            drain16(ngrp - 1)
            pltpu.async_copy(rows_v, out_hbm.at[pl.ds(base + ch * csz, csz)],
                             sem_i).wait()
            return 0

        lax.fori_loop(0, nch, chunk, 0)

    return gk(table, idx)


def kernel(x, tags, mask, embed_table, W, b, transitions):
    idx = jnp.transpose(x).reshape(-1)
    g = _sc_gather(embed_table, idx)
    out = pl.pallas_call(
        _crf_body,
        out_shape=jax.ShapeDtypeStruct((1, 1), jnp.float32),
    )(g, jnp.transpose(tags), W, b.reshape(NL, 1), transitions,
      jnp.transpose(transitions))
    return out[0, 0]


# speedup vs baseline: 3.2277x; 1.0615x over previous
import functools

import jax
import jax.numpy as jnp
from jax import lax
from jax.experimental import pallas as pl
from jax.experimental.pallas import tpu as pltpu
from jax.experimental.pallas import tpu_sc as plsc

BATCH = 1024
SEQ = 50
EMB = 16
NL = 16
TOT = BATCH * SEQ


def _crf_body(g_ref, tags_ref, w_ref, b_ref, trans_ref, transT_ref, out_ref):
    Wm = w_ref[...]
    bias = b_ref[...]
    trans = trans_ref[...]
    transT = transT_ref[...]
    Et = jnp.exp(transT)
    lab_iota = lax.broadcasted_iota(jnp.int32, (NL, BATCH), 0)
    tr_start = transT[:, 0:1]
    tr_end = trans[:, 1:2]

    def emit(t):
        e = g_ref[pl.ds(t * BATCH, BATCH), :]
        em = lax.dot_general(Wm, e, (((1,), (1,)), ((), ())),
                             preferred_element_type=jnp.float32,
                             precision=lax.Precision.DEFAULT)
        return em + bias

    def selmask(t):
        tg = tags_ref[pl.ds(t, 1), :]
        return (lab_iota == tg).astype(jnp.float32)

    em0 = emit(0)
    sel0 = selmask(0)
    alphas0 = tr_start + em0
    acc0 = sel0 * (em0 + tr_start)

    def step(t, carry):
        alphas, acc, selp = carry
        em = emit(t)
        sel = selmask(t)
        m = jnp.max(alphas, axis=0, keepdims=True)
        p = jnp.exp(alphas - m)
        s = lax.dot_general(Et, p, (((1,), (0,)), ((), ())),
                            preferred_element_type=jnp.float32,
                            precision=lax.Precision.DEFAULT)
        alphas = em + m + jnp.log(s)
        tsel = lax.dot_general(transT, selp, (((1,), (0,)), ((), ())),
                               preferred_element_type=jnp.float32,
                               precision=lax.Precision.HIGHEST)
        acc = acc + sel * (em + tsel)
        return alphas, acc, sel

    alphas, acc, sel_last = lax.fori_loop(1, SEQ, step, (alphas0, acc0, sel0))
    acc = acc + sel_last * tr_end
    end = alphas + tr_end
    m = jnp.max(end, axis=0, keepdims=True)
    part = m + jnp.log(jnp.sum(jnp.exp(end - m), axis=0, keepdims=True))
    out_ref[...] = (jnp.sum(part) - jnp.sum(acc)).reshape(1, 1)


def _sc_gather(table, idx):
    info = plsc.get_sparse_core_info()
    nc, ns = info.num_cores, info.num_subcores
    nw = nc * ns
    bpw = TOT // nw
    csz = 400
    nch = bpw // csz
    ngrp = csz // 16

    mesh = plsc.VectorSubcoreMesh(core_axis_name="c", subcore_axis_name="s")

    @functools.partial(
        pl.kernel,
        mesh=mesh,
        out_type=jax.ShapeDtypeStruct((TOT, EMB), jnp.float32),
        scratch_types=[
            pltpu.VMEM((bpw,), jnp.int32),
            pltpu.VMEM((csz, EMB), jnp.float32),
            pltpu.SemaphoreType.DMA,
            pltpu.SemaphoreType.DMA,
        ],
    )
    def gk(table_hbm, idx_hbm, out_hbm, idx_v, rows_v, sem_i, sem_g):
        wid = lax.axis_index("s") * nc + lax.axis_index("c")
        base = wid * bpw
        pltpu.async_copy(idx_hbm.at[pl.ds(base, bpw)], idx_v, sem_i).wait()

        def issue16(ch, v):
            vec = idx_v[pl.ds(ch * csz + v * 16, 16)]
            for j in range(16):
                pltpu.make_async_copy(
                    table_hbm.at[pl.ds(vec[j], 1), :],
                    rows_v.at[pl.ds(v * 16 + j, 1), :],
                    sem_g,
                ).start()

        def drain16(v):
            for j in range(16):
                pltpu.make_async_copy(
                    table_hbm.at[pl.ds(0, 1), :],
                    rows_v.at[pl.ds(v * 16 + j, 1), :],
                    sem_g,
                ).wait()

        def chunk(ch, _):
            issue16(ch, 0)

            def grp(v, _):
                issue16(ch, v)
                drain16(v - 1)
                return 0

            lax.fori_loop(1, ngrp, grp, 0)
            drain16(ngrp - 1)
            pltpu.async_copy(rows_v, out_hbm.at[pl.ds(base + ch * csz, csz)],
                             sem_i).wait()
            return 0

        lax.fori_loop(0, nch, chunk, 0)

    return gk(table, idx)


def kernel(x, tags, mask, embed_table, W, b, transitions):
    idx = jnp.transpose(x).reshape(-1)
    g = _sc_gather(embed_table, idx)
    out = pl.pallas_call(
        _crf_body,
        out_shape=jax.ShapeDtypeStruct((1, 1), jnp.float32),
    )(g, jnp.transpose(tags), W, b.reshape(NL, 1), transitions,
      jnp.transpose(transitions))
    return out[0, 0]
